# baseline (speedup 1.0000x reference)
import functools

import jax
import jax.numpy as jnp
from jax.experimental import pallas as pl
from jax.experimental.pallas import tpu as pltpu

_N_FEATURES = 64
_N_CENTROIDS = 1024
_TOKENS_PER_STEP = 1024


def _vq_kernel(x_ref, c_ref, out_ref, loss_ref, *, n_steps, n_total):
    b = pl.program_id(0)
    xb = x_ref[0]
    cents = c_ref[...]

    mm = jax.lax.dot_general(
        xb, cents, (((0,), (0,)), ((), ())),
        preferred_element_type=jnp.float32,
    )
    xnorm = jnp.sum(xb * xb, axis=0)
    cnorm = jnp.sum(cents * cents, axis=0)
    dist = (xnorm[:, None] - 2.0 * mm) + cnorm[None, :]

    lane_iota = jax.lax.broadcasted_iota(
        jnp.int32, (_TOKENS_PER_STEP, _N_CENTROIDS), 1
    )
    m = jnp.min(dist, axis=1)
    masked = jnp.where(dist == m[:, None], lane_iota, _N_CENTROIDS)
    idx = jnp.min(masked, axis=1)

    one_hot = (lane_iota == idx[:, None]).astype(jnp.float32)

    q = jax.lax.dot_general(
        cents, one_hot, (((1,), (1,)), ((), ())),
        preferred_element_type=jnp.float32,
    )
    out_ref[0] = q

    partial = jnp.sum(m)

    @pl.when(b == 0)
    def _init():
        loss_ref[0, 0] = partial

    @pl.when(b != 0)
    def _acc():
        loss_ref[0, 0] += partial

    @pl.when(b == n_steps - 1)
    def _finish():
        loss_ref[0, 0] = loss_ref[0, 0] / n_total


@jax.jit
def kernel(x, centroids):
    b, c, w, h = x.shape
    x3 = x.reshape(b, c, w * h)
    n_total = float(b * c * w * h)

    out, loss = pl.pallas_call(
        functools.partial(_vq_kernel, n_steps=b, n_total=n_total),
        grid=(b,),
        in_specs=[
            pl.BlockSpec((1, c, w * h), lambda i: (i, 0, 0)),
            pl.BlockSpec((c, _N_CENTROIDS), lambda i: (0, 0)),
        ],
        out_specs=[
            pl.BlockSpec((1, c, w * h), lambda i: (i, 0, 0)),
            pl.BlockSpec(memory_space=pltpu.SMEM),
        ],
        out_shape=[
            jax.ShapeDtypeStruct((b, c, w * h), jnp.float32),
            jax.ShapeDtypeStruct((1, 1), jnp.float32),
        ],
    )(x3, centroids)

    return out.reshape(b, c, w, h), loss[0, 0]


# speedup vs baseline: 1.0778x; 1.0778x over previous
import functools

import jax
import jax.numpy as jnp
from jax.experimental import pallas as pl
from jax.experimental.pallas import tpu as pltpu

_N_FEATURES = 64
_N_CENTROIDS = 1024
_TOKENS_PER_STEP = 1024


def _vq_kernel(x_ref, c_ref, out_ref, loss_ref, *, n_steps, n_total):
    b = pl.program_id(0)
    xb = x_ref[0]
    cents = c_ref[...]

    mm = jax.lax.dot_general(
        xb, cents, (((0,), (0,)), ((), ())),
        preferred_element_type=jnp.float32,
    )
    xnorm = jnp.sum(xb * xb, axis=0)
    cnorm = jnp.sum(cents * cents, axis=0)
    dist = (xnorm[:, None] - 2.0 * mm) + cnorm[None, :]

    lane_iota = jax.lax.broadcasted_iota(
        jnp.int32, (_TOKENS_PER_STEP, _N_CENTROIDS), 1
    ).astype(jnp.float32)
    m = jnp.min(dist, axis=1)
    masked = jnp.where(dist == m[:, None], lane_iota, float(_N_CENTROIDS))
    idx = jnp.min(masked, axis=1)

    one_hot = (lane_iota == idx[:, None]).astype(jnp.float32)

    q = jax.lax.dot_general(
        cents, one_hot, (((1,), (1,)), ((), ())),
        preferred_element_type=jnp.float32,
    )
    out_ref[0] = q

    partial = jnp.sum(m)

    @pl.when(b == 0)
    def _init():
        loss_ref[0, 0] = partial

    @pl.when(b != 0)
    def _acc():
        loss_ref[0, 0] += partial

    @pl.when(b == n_steps - 1)
    def _finish():
        loss_ref[0, 0] = loss_ref[0, 0] / n_total


@jax.jit
def kernel(x, centroids):
    b, c, w, h = x.shape
    x3 = x.reshape(b, c, w * h)
    n_total = float(b * c * w * h)

    out, loss = pl.pallas_call(
        functools.partial(_vq_kernel, n_steps=b, n_total=n_total),
        grid=(b,),
        in_specs=[
            pl.BlockSpec((1, c, w * h), lambda i: (i, 0, 0)),
            pl.BlockSpec((c, _N_CENTROIDS), lambda i: (0, 0)),
        ],
        out_specs=[
            pl.BlockSpec((1, c, w * h), lambda i: (i, 0, 0)),
            pl.BlockSpec(memory_space=pltpu.SMEM),
        ],
        out_shape=[
            jax.ShapeDtypeStruct((b, c, w * h), jnp.float32),
            jax.ShapeDtypeStruct((1, 1), jnp.float32),
        ],
    )(x3, centroids)

    return out.reshape(b, c, w, h), loss[0, 0]
